# Initial kernel scaffold; baseline (speedup 1.0000x reference)
import functools

import jax
import jax.numpy as jnp
from jax import lax
from jax.experimental import pallas as pl
from jax.experimental.pallas import tpu as pltpu
from jax.experimental.pallas import tpu_sc as plsc

_N = 10000
_E = 320000
_DIN = 128
_DHID = 16
_NCLS = 10

_NPAD = 10240
_RB = 1024
_GRID = _NPAD // _RB

_NCORES = 2
_NSUB = 16
_NW = _NCORES * _NSUB
_EPW = _E // _NW
_CH = 80
_NCH = _EPW // _CH
_RPT = _NPAD // _NSUB
_DEGW = 8


def _mesh():
    return plsc.VectorSubcoreMesh(
        core_axis_name="c", subcore_axis_name="s",
        num_cores=_NCORES, num_subcores=_NSUB)


def _sc_degree(dst2d, ones, zeros8):
    @functools.partial(
        pl.kernel, mesh=_mesh(),
        out_type=jax.ShapeDtypeStruct((_NCORES, _NPAD, _DEGW), jnp.float32),
        scratch_types=[
            pltpu.VMEM((_NCH, _CH), jnp.int32),
            pltpu.VMEM((_CH, _DEGW), jnp.float32),
            pltpu.VMEM_SHARED((_NPAD, _DEGW), jnp.float32),
        ])
    def deg_kernel(dst_hbm, ones_hbm, z_hbm, out_hbm, didx, ones_v, acc):
        cid = lax.axis_index("c")
        sid = lax.axis_index("s")
        wid = sid * _NCORES + cid
        pltpu.sync_copy(z_hbm, acc.at[pl.ds(sid * _RPT, _RPT)])
        pltpu.sync_copy(ones_hbm, ones_v)
        pltpu.sync_copy(dst_hbm.at[pl.ds(wid * _NCH, _NCH)], didx)
        plsc.subcore_barrier()

        def chunk(j, c):
            pltpu.sync_copy(ones_v, acc.at[didx.at[j]], add=True)
            return c

        lax.fori_loop(0, _NCH, chunk, None)
        plsc.subcore_barrier()
        pltpu.sync_copy(acc.at[pl.ds(sid * _RPT, _RPT)],
                        out_hbm.at[cid, pl.ds(sid * _RPT, _RPT)])

    return deg_kernel(dst2d, ones, zeros8)


def _sc_scatter(hh, src2d, dst2d, zeros16):
    @functools.partial(
        pl.kernel, mesh=_mesh(),
        out_type=jax.ShapeDtypeStruct((_NCORES, _NPAD, _DHID), jnp.float32),
        scratch_types=[
            pltpu.VMEM((_NCH, _CH), jnp.int32),
            pltpu.VMEM((_NCH, _CH), jnp.int32),
            pltpu.VMEM((_CH, _DHID), jnp.float32),
            pltpu.VMEM_SHARED((_NPAD, _DHID), jnp.float32),
            pltpu.SemaphoreType.DMA,
        ])
    def scat_kernel(hh_hbm, src_hbm, dst_hbm, z_hbm, out_hbm,
                    sidx, didx, rows, acc, sem):
        cid = lax.axis_index("c")
        sid = lax.axis_index("s")
        wid = sid * _NCORES + cid
        pltpu.sync_copy(z_hbm, acc.at[pl.ds(sid * _RPT, _RPT)])
        pltpu.sync_copy(src_hbm.at[pl.ds(wid * _NCH, _NCH)], sidx)
        pltpu.sync_copy(dst_hbm.at[pl.ds(wid * _NCH, _NCH)], didx)
        plsc.subcore_barrier()

        def chunk(j, c):
            pltpu.async_copy(hh_hbm.at[sidx.at[j]], rows, sem).wait()
            pltpu.sync_copy(rows, acc.at[didx.at[j]], add=True)
            return c

        lax.fori_loop(0, _NCH, chunk, None)
        plsc.subcore_barrier()
        pltpu.sync_copy(acc.at[pl.ds(sid * _RPT, _RPT)],
                        out_hbm.at[cid, pl.ds(sid * _RPT, _RPT)])

    return scat_kernel(hh, src2d, dst2d, zeros16)


def _dinv_block(deg_ref):
    dsum = deg_ref[0] + deg_ref[1] + 1.0
    return lax.rsqrt(jnp.maximum(dsum, 1.0))[:, :1]


def _tc1_body(x_ref, w1_ref, deg_ref, o_ref):
    dinv = _dinv_block(deg_ref)
    h = jnp.dot(x_ref[...], w1_ref[...], preferred_element_type=jnp.float32)
    o_ref[...] = h * dinv


def _tc1(x_pad, W1, deg_parts):
    return pl.pallas_call(
        _tc1_body,
        grid=(_GRID,),
        in_specs=[
            pl.BlockSpec((_RB, _DIN), lambda i: (i, 0)),
            pl.BlockSpec((_DIN, _DHID), lambda i: (0, 0)),
            pl.BlockSpec((_NCORES, _RB, _DEGW), lambda i: (0, i, 0)),
        ],
        out_specs=pl.BlockSpec((_RB, _DHID), lambda i: (i, 0)),
        out_shape=jax.ShapeDtypeStruct((_NPAD, _DHID), jnp.float32),
    )(x_pad, W1, deg_parts)


def _tc2_body(acc_ref, hh_ref, deg_ref, w2_ref, b1_ref, o_ref):
    dinv = _dinv_block(deg_ref)
    s = acc_ref[0] + acc_ref[1] + hh_ref[...]
    h1 = jnp.maximum(s * dinv + b1_ref[...], 0.0)
    h2 = jnp.dot(h1, w2_ref[...], preferred_element_type=jnp.float32)
    o_ref[...] = h2 * dinv


def _tc2(acc1, hh, deg_parts, W2p, b1r):
    return pl.pallas_call(
        _tc2_body,
        grid=(_GRID,),
        in_specs=[
            pl.BlockSpec((_NCORES, _RB, _DHID), lambda i: (0, i, 0)),
            pl.BlockSpec((_RB, _DHID), lambda i: (i, 0)),
            pl.BlockSpec((_NCORES, _RB, _DEGW), lambda i: (0, i, 0)),
            pl.BlockSpec((_DHID, _DHID), lambda i: (0, 0)),
            pl.BlockSpec((1, _DHID), lambda i: (0, 0)),
        ],
        out_specs=pl.BlockSpec((_RB, _DHID), lambda i: (i, 0)),
        out_shape=jax.ShapeDtypeStruct((_NPAD, _DHID), jnp.float32),
    )(acc1, hh, deg_parts, W2p, b1r)


def _tc3_body(acc_ref, hh_ref, deg_ref, b2_ref, o_ref):
    dinv = _dinv_block(deg_ref)
    s = acc_ref[0] + acc_ref[1] + hh_ref[...]
    z = s * dinv + b2_ref[...]
    col = lax.broadcasted_iota(jnp.int32, (_RB, _DHID), 1)
    mask = col < _NCLS
    neg = jnp.full_like(z, -3.0e38)
    m = jnp.max(jnp.where(mask, z, neg), axis=1, keepdims=True)
    e = jnp.where(mask, jnp.exp(z - m), 0.0)
    lse = jnp.log(jnp.sum(e, axis=1, keepdims=True))
    o_ref[...] = z - m - lse


def _tc3(acc2, hh2, deg_parts, b2r):
    return pl.pallas_call(
        _tc3_body,
        grid=(_GRID,),
        in_specs=[
            pl.BlockSpec((_NCORES, _RB, _DHID), lambda i: (0, i, 0)),
            pl.BlockSpec((_RB, _DHID), lambda i: (i, 0)),
            pl.BlockSpec((_NCORES, _RB, _DEGW), lambda i: (0, i, 0)),
            pl.BlockSpec((1, _DHID), lambda i: (0, 0)),
        ],
        out_specs=pl.BlockSpec((_RB, _DHID), lambda i: (i, 0)),
        out_shape=jax.ShapeDtypeStruct((_NPAD, _DHID), jnp.float32),
    )(acc2, hh2, deg_parts, b2r)


def kernel(x, edge_index, W1, b1, W2, b2):
    src2d = edge_index[0].astype(jnp.int32).reshape(_NW * _NCH, _CH)
    dst2d = edge_index[1].astype(jnp.int32).reshape(_NW * _NCH, _CH)
    x_pad = jnp.pad(x, ((0, _NPAD - _N), (0, 0)))
    W2p = jnp.pad(W2, ((0, 0), (0, _DHID - _NCLS)))
    b1r = b1.reshape(1, _DHID)
    b2r = jnp.pad(b2, (0, _DHID - _NCLS)).reshape(1, _DHID)
    ones = jnp.ones((_CH, _DEGW), jnp.float32)
    zeros8 = jnp.zeros((_RPT, _DEGW), jnp.float32)
    zeros16 = jnp.zeros((_RPT, _DHID), jnp.float32)

    deg_parts = _sc_degree(dst2d, ones, zeros8)
    hh = _tc1(x_pad, W1, deg_parts)
    acc1 = _sc_scatter(hh, src2d, dst2d, zeros16)
    hh2 = _tc2(acc1, hh, deg_parts, W2p, b1r)
    acc2 = _sc_scatter(hh2, src2d, dst2d, zeros16)
    outp = _tc3(acc2, hh2, deg_parts, b2r)
    return outp[:_N, :_NCLS]


# speedup vs baseline: 28.7208x; 28.7208x over previous
import functools

import jax
import jax.numpy as jnp
from jax import lax
from jax.experimental import pallas as pl
from jax.experimental.pallas import tpu as pltpu
from jax.experimental.pallas import tpu_sc as plsc

_N = 10000
_E = 320000
_DIN = 128
_DHID = 16
_NCLS = 10

_NPAD = 10240
_RB = 1024
_GRID = _NPAD // _RB

_NCORES = 2
_NSUB = 16
_NW = _NCORES * _NSUB
_EPW = _E // _NW
_CH = 80
_NCH = _EPW // _CH
_RPT = _NPAD // _NSUB
_DEGW = 8


def _mesh():
    return plsc.VectorSubcoreMesh(
        core_axis_name="c", subcore_axis_name="s",
        num_cores=_NCORES, num_subcores=_NSUB)


def _sc_degree(dst2d, ones, zeros8):
    @functools.partial(
        pl.kernel, mesh=_mesh(),
        compiler_params=pltpu.CompilerParams(use_tc_tiling_on_sc=False),
        out_type=jax.ShapeDtypeStruct((_NCORES, _NPAD, _DEGW), jnp.float32),
        scratch_types=[
            pltpu.VMEM((_NCH, _CH), jnp.int32),
            pltpu.VMEM((_CH, _DEGW), jnp.float32),
            pltpu.VMEM_SHARED((_NPAD, _DEGW), jnp.float32),
        ])
    def deg_kernel(dst_hbm, ones_hbm, z_hbm, out_hbm, didx, ones_v, acc):
        cid = lax.axis_index("c")
        sid = lax.axis_index("s")
        wid = sid * _NCORES + cid
        pltpu.sync_copy(z_hbm, acc.at[pl.ds(sid * _RPT, _RPT)])
        pltpu.sync_copy(ones_hbm, ones_v)
        pltpu.sync_copy(dst_hbm.at[pl.ds(wid * _NCH, _NCH)], didx)
        plsc.subcore_barrier()

        def chunk(j, c):
            pltpu.sync_copy(ones_v, acc.at[didx.at[j]], add=True)
            return c

        lax.fori_loop(0, _NCH, chunk, None)
        plsc.subcore_barrier()
        pltpu.sync_copy(acc.at[pl.ds(sid * _RPT, _RPT)],
                        out_hbm.at[cid, pl.ds(sid * _RPT, _RPT)])

    return deg_kernel(dst2d, ones, zeros8)


def _sc_scatter(hh, src2d, dst2d, zeros16):
    @functools.partial(
        pl.kernel, mesh=_mesh(),
        compiler_params=pltpu.CompilerParams(use_tc_tiling_on_sc=False),
        out_type=jax.ShapeDtypeStruct((_NCORES, _NPAD, _DHID), jnp.float32),
        scratch_types=[
            pltpu.VMEM((_NCH, _CH), jnp.int32),
            pltpu.VMEM((_NCH, _CH), jnp.int32),
            pltpu.VMEM((_CH, _DHID), jnp.float32),
            pltpu.VMEM_SHARED((_NPAD, _DHID), jnp.float32),
            pltpu.SemaphoreType.DMA,
        ])
    def scat_kernel(hh_hbm, src_hbm, dst_hbm, z_hbm, out_hbm,
                    sidx, didx, rows, acc, sem):
        cid = lax.axis_index("c")
        sid = lax.axis_index("s")
        wid = sid * _NCORES + cid
        pltpu.sync_copy(z_hbm, acc.at[pl.ds(sid * _RPT, _RPT)])
        pltpu.sync_copy(src_hbm.at[pl.ds(wid * _NCH, _NCH)], sidx)
        pltpu.sync_copy(dst_hbm.at[pl.ds(wid * _NCH, _NCH)], didx)
        plsc.subcore_barrier()

        def chunk(j, c):
            pltpu.async_copy(hh_hbm.at[sidx.at[j]], rows, sem).wait()
            pltpu.sync_copy(rows, acc.at[didx.at[j]], add=True)
            return c

        lax.fori_loop(0, _NCH, chunk, None)
        plsc.subcore_barrier()
        pltpu.sync_copy(acc.at[pl.ds(sid * _RPT, _RPT)],
                        out_hbm.at[cid, pl.ds(sid * _RPT, _RPT)])

    return scat_kernel(hh, src2d, dst2d, zeros16)


def _dinv_block(deg_ref):
    dsum = deg_ref[0] + deg_ref[1] + 1.0
    return lax.rsqrt(jnp.maximum(dsum, 1.0))[:, :1]


def _tc1_body(x_ref, w1_ref, deg_ref, o_ref):
    dinv = _dinv_block(deg_ref)
    h = jnp.dot(x_ref[...], w1_ref[...], preferred_element_type=jnp.float32)
    o_ref[...] = h * dinv


def _tc1(x_pad, W1, deg_parts):
    return pl.pallas_call(
        _tc1_body,
        grid=(_GRID,),
        in_specs=[
            pl.BlockSpec((_RB, _DIN), lambda i: (i, 0)),
            pl.BlockSpec((_DIN, _DHID), lambda i: (0, 0)),
            pl.BlockSpec((_NCORES, _RB, _DEGW), lambda i: (0, i, 0)),
        ],
        out_specs=pl.BlockSpec((_RB, _DHID), lambda i: (i, 0)),
        out_shape=jax.ShapeDtypeStruct((_NPAD, _DHID), jnp.float32),
    )(x_pad, W1, deg_parts)


def _tc2_body(acc_ref, hh_ref, deg_ref, w2_ref, b1_ref, o_ref):
    dinv = _dinv_block(deg_ref)
    s = acc_ref[0] + acc_ref[1] + hh_ref[...]
    h1 = jnp.maximum(s * dinv + b1_ref[...], 0.0)
    h2 = jnp.dot(h1, w2_ref[...], preferred_element_type=jnp.float32)
    o_ref[...] = h2 * dinv


def _tc2(acc1, hh, deg_parts, W2p, b1r):
    return pl.pallas_call(
        _tc2_body,
        grid=(_GRID,),
        in_specs=[
            pl.BlockSpec((_NCORES, _RB, _DHID), lambda i: (0, i, 0)),
            pl.BlockSpec((_RB, _DHID), lambda i: (i, 0)),
            pl.BlockSpec((_NCORES, _RB, _DEGW), lambda i: (0, i, 0)),
            pl.BlockSpec((_DHID, _DHID), lambda i: (0, 0)),
            pl.BlockSpec((1, _DHID), lambda i: (0, 0)),
        ],
        out_specs=pl.BlockSpec((_RB, _DHID), lambda i: (i, 0)),
        out_shape=jax.ShapeDtypeStruct((_NPAD, _DHID), jnp.float32),
    )(acc1, hh, deg_parts, W2p, b1r)


def _tc3_body(acc_ref, hh_ref, deg_ref, b2_ref, o_ref):
    dinv = _dinv_block(deg_ref)
    s = acc_ref[0] + acc_ref[1] + hh_ref[...]
    z = s * dinv + b2_ref[...]
    col = lax.broadcasted_iota(jnp.int32, (_RB, _DHID), 1)
    mask = col < _NCLS
    neg = jnp.full_like(z, -3.0e38)
    m = jnp.max(jnp.where(mask, z, neg), axis=1, keepdims=True)
    e = jnp.where(mask, jnp.exp(z - m), 0.0)
    lse = jnp.log(jnp.sum(e, axis=1, keepdims=True))
    o_ref[...] = z - m - lse


def _tc3(acc2, hh2, deg_parts, b2r):
    return pl.pallas_call(
        _tc3_body,
        grid=(_GRID,),
        in_specs=[
            pl.BlockSpec((_NCORES, _RB, _DHID), lambda i: (0, i, 0)),
            pl.BlockSpec((_RB, _DHID), lambda i: (i, 0)),
            pl.BlockSpec((_NCORES, _RB, _DEGW), lambda i: (0, i, 0)),
            pl.BlockSpec((1, _DHID), lambda i: (0, 0)),
        ],
        out_specs=pl.BlockSpec((_RB, _DHID), lambda i: (i, 0)),
        out_shape=jax.ShapeDtypeStruct((_NPAD, _DHID), jnp.float32),
    )(acc2, hh2, deg_parts, b2r)


def kernel(x, edge_index, W1, b1, W2, b2):
    src2d = edge_index[0].astype(jnp.int32).reshape(_NW * _NCH, _CH)
    dst2d = edge_index[1].astype(jnp.int32).reshape(_NW * _NCH, _CH)
    x_pad = jnp.pad(x, ((0, _NPAD - _N), (0, 0)))
    W2p = jnp.pad(W2, ((0, 0), (0, _DHID - _NCLS)))
    b1r = b1.reshape(1, _DHID)
    b2r = jnp.pad(b2, (0, _DHID - _NCLS)).reshape(1, _DHID)
    ones = jnp.ones((_CH, _DEGW), jnp.float32)
    zeros8 = jnp.zeros((_RPT, _DEGW), jnp.float32)
    zeros16 = jnp.zeros((_RPT, _DHID), jnp.float32)

    deg_parts = _sc_degree(dst2d, ones, zeros8)
    hh = _tc1(x_pad, W1, deg_parts)
    acc1 = _sc_scatter(hh, src2d, dst2d, zeros16)
    hh2 = _tc2(acc1, hh, deg_parts, W2p, b1r)
    acc2 = _sc_scatter(hh2, src2d, dst2d, zeros16)
    outp = _tc3(acc2, hh2, deg_parts, b2r)
    return outp[:_N, :_NCLS]


# speedup vs baseline: 40.1204x; 1.3969x over previous
import functools

import jax
import jax.numpy as jnp
from jax import lax
from jax.experimental import pallas as pl
from jax.experimental.pallas import tpu as pltpu
from jax.experimental.pallas import tpu_sc as plsc

_N = 10000
_E = 320000
_DIN = 128
_DHID = 16
_NCLS = 10

_NPAD = 10240
_RB = 1024
_GRID = _NPAD // _RB

_NCORES = 2
_NSUB = 16
_NW = _NCORES * _NSUB
_EPW = _E // _NW
_CH = 80
_NCH = _EPW // _CH
_RPT = _NPAD // _NSUB
_DEGW = 8


def _mesh():
    return plsc.VectorSubcoreMesh(
        core_axis_name="c", subcore_axis_name="s",
        num_cores=_NCORES, num_subcores=_NSUB)


def _sc_degree(dst2d, ones, zeros8):
    @functools.partial(
        pl.kernel, mesh=_mesh(),
        compiler_params=pltpu.CompilerParams(use_tc_tiling_on_sc=False),
        out_type=jax.ShapeDtypeStruct((_NCORES, _NPAD, _DEGW), jnp.float32),
        scratch_types=[
            pltpu.VMEM((_NCH, _CH), jnp.int32),
            pltpu.VMEM((_CH, _DEGW), jnp.float32),
            pltpu.VMEM_SHARED((_NPAD, _DEGW), jnp.float32),
        ])
    def deg_kernel(dst_hbm, ones_hbm, z_hbm, out_hbm, didx, ones_v, acc):
        cid = lax.axis_index("c")
        sid = lax.axis_index("s")
        wid = sid * _NCORES + cid
        pltpu.sync_copy(z_hbm, acc.at[pl.ds(sid * _RPT, _RPT)])
        pltpu.sync_copy(ones_hbm, ones_v)
        pltpu.sync_copy(dst_hbm.at[pl.ds(wid * _NCH, _NCH)], didx)
        plsc.subcore_barrier()

        def chunk(j, c):
            pltpu.sync_copy(ones_v, acc.at[didx.at[j]], add=True)
            return c

        lax.fori_loop(0, _NCH, chunk, None)
        plsc.subcore_barrier()
        pltpu.sync_copy(acc.at[pl.ds(sid * _RPT, _RPT)],
                        out_hbm.at[cid, pl.ds(sid * _RPT, _RPT)])

    return deg_kernel(dst2d, ones, zeros8)


def _sc_scatter(hh, src2d, dst2d, zeros16):
    @functools.partial(
        pl.kernel, mesh=_mesh(),
        compiler_params=pltpu.CompilerParams(use_tc_tiling_on_sc=False),
        out_type=jax.ShapeDtypeStruct((_NCORES, _NPAD, _DHID), jnp.float32),
        scratch_types=[
            pltpu.VMEM((_NCH, _CH), jnp.int32),
            pltpu.VMEM((_NCH, _CH), jnp.int32),
            pltpu.VMEM((2, _CH, _DHID), jnp.float32),
            pltpu.VMEM_SHARED((_NPAD, _DHID), jnp.float32),
            pltpu.SemaphoreType.DMA,
            pltpu.SemaphoreType.DMA,
        ])
    def scat_kernel(hh_hbm, src_hbm, dst_hbm, z_hbm, out_hbm,
                    sidx, didx, rows, acc, sem_a, sem_b):
        cid = lax.axis_index("c")
        sid = lax.axis_index("s")
        wid = sid * _NCORES + cid
        pltpu.sync_copy(z_hbm, acc.at[pl.ds(sid * _RPT, _RPT)])
        pltpu.sync_copy(src_hbm.at[pl.ds(wid * _NCH, _NCH)], sidx)
        pltpu.sync_copy(dst_hbm.at[pl.ds(wid * _NCH, _NCH)], didx)
        plsc.subcore_barrier()

        pltpu.async_copy(hh_hbm.at[sidx.at[0]], rows.at[0], sem_a)

        def pair(j, c):
            e = 2 * j
            pltpu.async_copy(hh_hbm.at[sidx.at[e + 1]], rows.at[1], sem_b)
            pltpu.make_async_copy(hh_hbm.at[sidx.at[e]], rows.at[0],
                                  sem_a).wait()
            pltpu.sync_copy(rows.at[0], acc.at[didx.at[e]], add=True)
            pltpu.async_copy(hh_hbm.at[sidx.at[e + 2]], rows.at[0], sem_a)
            pltpu.make_async_copy(hh_hbm.at[sidx.at[e + 1]], rows.at[1],
                                  sem_b).wait()
            pltpu.sync_copy(rows.at[1], acc.at[didx.at[e + 1]], add=True)
            return c

        lax.fori_loop(0, (_NCH - 1) // 2, pair, None)
        last = _NCH - 1
        pltpu.make_async_copy(hh_hbm.at[sidx.at[last]], rows.at[0],
                              sem_a).wait()
        pltpu.sync_copy(rows.at[0], acc.at[didx.at[last]], add=True)
        plsc.subcore_barrier()
        pltpu.sync_copy(acc.at[pl.ds(sid * _RPT, _RPT)],
                        out_hbm.at[cid, pl.ds(sid * _RPT, _RPT)])

    return scat_kernel(hh, src2d, dst2d, zeros16)


def _dinv_block(deg_ref):
    dsum = deg_ref[0] + deg_ref[1] + 1.0
    return lax.rsqrt(jnp.maximum(dsum, 1.0))[:, :1]


def _tc1_body(x_ref, w1_ref, deg_ref, o_ref):
    dinv = _dinv_block(deg_ref)
    h = jnp.dot(x_ref[...], w1_ref[...], preferred_element_type=jnp.float32)
    o_ref[...] = h * dinv


def _tc1(x, W1, deg_parts):
    return pl.pallas_call(
        _tc1_body,
        grid=(_GRID,),
        in_specs=[
            pl.BlockSpec((_RB, _DIN), lambda i: (i, 0)),
            pl.BlockSpec((_DIN, _DHID), lambda i: (0, 0)),
            pl.BlockSpec((_NCORES, _RB, _DEGW), lambda i: (0, i, 0)),
        ],
        out_specs=pl.BlockSpec((_RB, _DHID), lambda i: (i, 0)),
        out_shape=jax.ShapeDtypeStruct((_N, _DHID), jnp.float32),
    )(x, W1, deg_parts)


def _tc2_body(acc_ref, hh_ref, deg_ref, w2_ref, b1_ref, o_ref):
    dinv = _dinv_block(deg_ref)
    s = acc_ref[0] + acc_ref[1] + hh_ref[...]
    h1 = jnp.maximum(s * dinv + b1_ref[...], 0.0)
    h2 = jnp.dot(h1, w2_ref[...], preferred_element_type=jnp.float32)
    o_ref[...] = h2 * dinv


def _tc2(acc1, hh, deg_parts, W2p, b1r):
    return pl.pallas_call(
        _tc2_body,
        grid=(_GRID,),
        in_specs=[
            pl.BlockSpec((_NCORES, _RB, _DHID), lambda i: (0, i, 0)),
            pl.BlockSpec((_RB, _DHID), lambda i: (i, 0)),
            pl.BlockSpec((_NCORES, _RB, _DEGW), lambda i: (0, i, 0)),
            pl.BlockSpec((_DHID, _DHID), lambda i: (0, 0)),
            pl.BlockSpec((1, _DHID), lambda i: (0, 0)),
        ],
        out_specs=pl.BlockSpec((_RB, _DHID), lambda i: (i, 0)),
        out_shape=jax.ShapeDtypeStruct((_N, _DHID), jnp.float32),
    )(acc1, hh, deg_parts, W2p, b1r)


def _tc3_body(acc_ref, hh_ref, deg_ref, b2_ref, o_ref):
    dinv = _dinv_block(deg_ref)
    s = acc_ref[0] + acc_ref[1] + hh_ref[...]
    z = s * dinv + b2_ref[...]
    col = lax.broadcasted_iota(jnp.int32, (_RB, _DHID), 1)
    mask = col < _NCLS
    neg = jnp.full_like(z, -3.0e38)
    m = jnp.max(jnp.where(mask, z, neg), axis=1, keepdims=True)
    e = jnp.where(mask, jnp.exp(z - m), 0.0)
    lse = jnp.log(jnp.sum(e, axis=1, keepdims=True))
    o_ref[...] = (z - m - lse)[:, :_NCLS]


def _tc3(acc2, hh2, deg_parts, b2r):
    return pl.pallas_call(
        _tc3_body,
        grid=(_GRID,),
        in_specs=[
            pl.BlockSpec((_NCORES, _RB, _DHID), lambda i: (0, i, 0)),
            pl.BlockSpec((_RB, _DHID), lambda i: (i, 0)),
            pl.BlockSpec((_NCORES, _RB, _DEGW), lambda i: (0, i, 0)),
            pl.BlockSpec((1, _DHID), lambda i: (0, 0)),
        ],
        out_specs=pl.BlockSpec((_RB, _NCLS), lambda i: (i, 0)),
        out_shape=jax.ShapeDtypeStruct((_N, _NCLS), jnp.float32),
    )(acc2, hh2, deg_parts, b2r)


def kernel(x, edge_index, W1, b1, W2, b2):
    src2d = edge_index[0].astype(jnp.int32).reshape(_NW * _NCH, _CH)
    dst2d = edge_index[1].astype(jnp.int32).reshape(_NW * _NCH, _CH)
    W2p = jnp.pad(W2, ((0, 0), (0, _DHID - _NCLS)))
    b1r = b1.reshape(1, _DHID)
    b2r = jnp.pad(b2, (0, _DHID - _NCLS)).reshape(1, _DHID)
    ones = jnp.ones((_CH, _DEGW), jnp.float32)
    zeros8 = jnp.zeros((_RPT, _DEGW), jnp.float32)
    zeros16 = jnp.zeros((_RPT, _DHID), jnp.float32)

    deg_parts = _sc_degree(dst2d, ones, zeros8)
    hh = _tc1(x, W1, deg_parts)
    acc1 = _sc_scatter(hh, src2d, dst2d, zeros16)
    hh2 = _tc2(acc1, hh, deg_parts, W2p, b1r)
    acc2 = _sc_scatter(hh2, src2d, dst2d, zeros16)
    return _tc3(acc2, hh2, deg_parts, b2r)
